# Initial kernel scaffold; baseline (speedup 1.0000x reference)
import jax
import jax.numpy as jnp
from jax.experimental import pallas as pl


def kernel(inputs, embedding):
    raise NotImplementedError("write your pallas kernel here")


# speedup vs baseline: 1.5549x; 1.5549x over previous
import functools

import jax
import jax.numpy as jnp
from jax import lax
from jax.experimental import pallas as pl
from jax.experimental.pallas import tpu as pltpu
from jax.experimental.pallas import tpu_sc as plsc

NUM_E = 1024
DIM = 64
N_ROWS = 32 * 576
N_HALF = N_ROWS // 2
ROW_BLK = 3072
CCHUNK = 512
GRID = N_HALF // ROW_BLK
LOSS_SCALE = 1.25 / (N_ROWS * DIM)

NW = 32
B_PER_W = N_HALF // NW
IDX_CHUNK = 72
CHUNKS_PER_W = B_PER_W // IDX_CHUNK


def _assign_body(x_ref, e_ref, idx_ref, loss_ref):
    x = x_ref[...]
    e = e_ref[...]
    xt = x.T
    mm2 = lax.dot_general(e, xt + xt, (((1,), (0,)), ((), ())),
                          preferred_element_type=jnp.float32)
    x2 = jnp.sum(xt * xt, axis=0, keepdims=True)
    e2 = jnp.sum(e * e, axis=1, keepdims=True)

    part = None
    for c in range(ROW_BLK // CCHUNK):
        cs = slice(c * CCHUNK, (c + 1) * CCHUNK)
        x2c = x2[:, cs]
        md8 = (x2c + e2[0:8]) - mm2[0:8, cs]
        iv8 = jnp.zeros(md8.shape, jnp.int32)
        for v in range(1, NUM_E // 8):
            dv = (x2c + e2[v * 8:(v + 1) * 8]) - mm2[v * 8:(v + 1) * 8, cs]
            upd = dv < md8
            md8 = jnp.minimum(md8, dv)
            iv8 = jnp.where(upd, v, iv8)

        code8 = iv8 * 8 + lax.broadcasted_iota(jnp.int32, md8.shape, 0)
        md = jnp.min(md8, axis=0, keepdims=True)
        idx_ref[pl.ds(c * CCHUNK, CCHUNK)] = jnp.min(
            jnp.where(md8 == md, code8, NUM_E), axis=0)
        psum = jnp.sum(md, axis=1, keepdims=True)
        part = psum if part is None else part + psum
    acc = jnp.broadcast_to(part, loss_ref.shape)

    @pl.when(pl.program_id(0) == 0)
    def _():
        loss_ref[...] = jnp.zeros_like(loss_ref)

    loss_ref[...] += acc

    @pl.when(pl.program_id(0) == GRID - 1)
    def _():
        loss_ref[...] = loss_ref[...] * LOSS_SCALE


def _make_assign(half):
    return pl.pallas_call(
        _assign_body,
        grid=(GRID,),
        in_specs=[
            pl.BlockSpec((ROW_BLK, DIM), lambda i: (i + half * GRID, 0)),
            pl.BlockSpec((NUM_E, DIM), lambda i: (0, 0)),
        ],
        out_specs=[
            pl.BlockSpec((ROW_BLK,), lambda i: (i,)),
            pl.BlockSpec((8, 128), lambda i: (0, 0)),
        ],
        out_shape=[
            jax.ShapeDtypeStruct((N_HALF,), jnp.int32),
            jax.ShapeDtypeStruct((8, 128), jnp.float32),
        ],
        compiler_params=pltpu.CompilerParams(
            dimension_semantics=("arbitrary",),
        ),
    )


_assign_halves = (_make_assign(0), _make_assign(1))


@functools.cache
def _make_gather_rows():
    @functools.partial(
        pl.kernel,
        mesh=plsc.VectorSubcoreMesh(core_axis_name="c", subcore_axis_name="s"),
        out_type=jax.ShapeDtypeStruct((N_HALF, DIM), jnp.float32),
        scratch_types=[
            pltpu.VMEM((B_PER_W,), jnp.int32),
            pltpu.VMEM((B_PER_W, DIM), jnp.float32),
            pltpu.SemaphoreType.DMA,
        ],
        compiler_params=pltpu.CompilerParams(use_tc_tiling_on_sc=False),
    )
    def _gather_rows(table_hbm, idx_hbm, out_hbm, idx_v, rows_v, sem):
        wid = lax.axis_index("s") * 2 + lax.axis_index("c")
        base = wid * B_PER_W
        pltpu.sync_copy(idx_hbm.at[pl.ds(base, B_PER_W)], idx_v)
        copies = []
        for j in range(CHUNKS_PER_W):
            copies.append(pltpu.async_copy(
                table_hbm.at[idx_v.at[pl.ds(j * IDX_CHUNK, IDX_CHUNK)]],
                rows_v.at[pl.ds(j * IDX_CHUNK, IDX_CHUNK)],
                sem))
        for c in copies:
            c.wait()
        pltpu.sync_copy(rows_v, out_hbm.at[pl.ds(base, B_PER_W)])

    return _gather_rows


def kernel(inputs, embedding):
    flat = inputs.reshape(N_ROWS, DIM)
    gather = _make_gather_rows()
    idx0, loss0 = _assign_halves[0](flat, embedding)
    q0 = gather(embedding, idx0)
    idx1, loss1 = _assign_halves[1](flat, embedding)
    q1 = gather(embedding, idx1)
    quantized = jnp.concatenate([q0, q1], axis=0).reshape(inputs.shape)
    return quantized, loss0[0, 0] + loss1[0, 0]


# speedup vs baseline: 1.7155x; 1.1033x over previous
import functools

import jax
import jax.numpy as jnp
from jax import lax
from jax.experimental import pallas as pl
from jax.experimental.pallas import tpu as pltpu
from jax.experimental.pallas import tpu_sc as plsc

NUM_E = 1024
DIM = 64
N_ROWS = 32 * 576
ROW_BLK = 6144
CCHUNK = 512
GRID = N_ROWS // ROW_BLK
LOSS_SCALE = 1.25 / (N_ROWS * DIM)

NW = 32
B_PER_W = N_ROWS // NW
IDX_CHUNK = 72
CHUNKS_PER_W = B_PER_W // IDX_CHUNK


def _assign_body(x_ref, e_ref, idx_ref, loss_ref):
    x = x_ref[...]
    e = e_ref[...]
    xt = x.T
    mm2 = lax.dot_general(e, xt + xt, (((1,), (0,)), ((), ())),
                          preferred_element_type=jnp.float32)
    x2 = jnp.sum(xt * xt, axis=0, keepdims=True)
    e2 = jnp.sum(e * e, axis=1, keepdims=True)

    part = None
    for c in range(ROW_BLK // CCHUNK):
        cs = slice(c * CCHUNK, (c + 1) * CCHUNK)
        x2c = x2[:, cs]
        md8 = (x2c + e2[0:8]) - mm2[0:8, cs]
        iv8 = jnp.zeros(md8.shape, jnp.int32)
        for v in range(1, NUM_E // 8):
            dv = (x2c + e2[v * 8:(v + 1) * 8]) - mm2[v * 8:(v + 1) * 8, cs]
            upd = dv < md8
            md8 = jnp.minimum(md8, dv)
            iv8 = jnp.where(upd, v, iv8)

        code8 = iv8 * 8 + lax.broadcasted_iota(jnp.int32, md8.shape, 0)
        md = jnp.min(md8, axis=0, keepdims=True)
        idx_ref[pl.ds(c * CCHUNK, CCHUNK)] = jnp.min(
            jnp.where(md8 == md, code8, NUM_E), axis=0)
        psum = jnp.sum(md, axis=1, keepdims=True)
        part = psum if part is None else part + psum
    acc = jnp.broadcast_to(part, loss_ref.shape)

    @pl.when(pl.program_id(0) == 0)
    def _():
        loss_ref[...] = jnp.zeros_like(loss_ref)

    loss_ref[...] += acc

    @pl.when(pl.program_id(0) == GRID - 1)
    def _():
        loss_ref[...] = loss_ref[...] * LOSS_SCALE


_assign = pl.pallas_call(
    _assign_body,
    grid=(GRID,),
    in_specs=[
        pl.BlockSpec((ROW_BLK, DIM), lambda i: (i, 0)),
        pl.BlockSpec((NUM_E, DIM), lambda i: (0, 0)),
    ],
    out_specs=[
        pl.BlockSpec((ROW_BLK,), lambda i: (i,)),
        pl.BlockSpec((8, 128), lambda i: (0, 0)),
    ],
    out_shape=[
        jax.ShapeDtypeStruct((N_ROWS,), jnp.int32),
        jax.ShapeDtypeStruct((8, 128), jnp.float32),
    ],
    compiler_params=pltpu.CompilerParams(
        dimension_semantics=("arbitrary",),
    ),
)


@functools.cache
def _make_gather_rows():
    @functools.partial(
        pl.kernel,
        mesh=plsc.VectorSubcoreMesh(core_axis_name="c", subcore_axis_name="s"),
        out_type=jax.ShapeDtypeStruct((N_ROWS, DIM), jnp.float32),
        scratch_types=[
            pltpu.VMEM((B_PER_W,), jnp.int32),
            pltpu.VMEM((B_PER_W, DIM), jnp.float32),
            pltpu.SemaphoreType.DMA,
        ],
        compiler_params=pltpu.CompilerParams(use_tc_tiling_on_sc=False),
    )
    def _gather_rows(table_hbm, idx_hbm, out_hbm, idx_v, rows_v, sem):
        wid = lax.axis_index("s") * 2 + lax.axis_index("c")
        base = wid * B_PER_W
        pltpu.sync_copy(idx_hbm.at[pl.ds(base, B_PER_W)], idx_v)
        pltpu.async_copy(table_hbm.at[idx_v], rows_v, sem).wait()
        pltpu.sync_copy(rows_v, out_hbm.at[pl.ds(base, B_PER_W)])

    return _gather_rows


def kernel(inputs, embedding):
    flat = inputs.reshape(N_ROWS, DIM)
    idx, loss_buf = _assign(flat, embedding)
    quantized = _make_gather_rows()(embedding, idx)
    return quantized.reshape(inputs.shape), loss_buf[0, 0]


# speedup vs baseline: 1.7176x; 1.0012x over previous
import functools

import jax
import jax.numpy as jnp
from jax import lax
from jax.experimental import pallas as pl
from jax.experimental.pallas import tpu as pltpu
from jax.experimental.pallas import tpu_sc as plsc

NUM_E = 1024
DIM = 64
N_ROWS = 32 * 576
ROW_BLK = 6144
CCHUNK = 512
GRID = N_ROWS // ROW_BLK
LOSS_SCALE = 1.25 / (N_ROWS * DIM)

NW = 32
B_PER_W = N_ROWS // NW
IDX_CHUNK = 72
CHUNKS_PER_W = B_PER_W // IDX_CHUNK


def _assign_body(x_ref, e_ref, idx_ref, loss_ref):
    x = x_ref[...]
    e = e_ref[...]
    xt = x.T
    mm2 = lax.dot_general(e, xt + xt, (((1,), (0,)), ((), ())),
                          preferred_element_type=jnp.float32)
    x2 = jnp.sum(xt * xt, axis=0, keepdims=True)
    e2 = jnp.sum(e * e, axis=1, keepdims=True)

    part = None
    for c in range(ROW_BLK // CCHUNK):
        cs = slice(c * CCHUNK, (c + 1) * CCHUNK)
        x2c = x2[:, cs]
        md8 = (x2c + e2[0:8]) - mm2[0:8, cs]
        iv8 = jnp.zeros(md8.shape, jnp.int32)
        for v in range(1, NUM_E // 8):
            dv = (x2c + e2[v * 8:(v + 1) * 8]) - mm2[v * 8:(v + 1) * 8, cs]
            upd = dv < md8
            md8 = jnp.minimum(md8, dv)
            iv8 = jnp.where(upd, v, iv8)

        code8 = iv8 * 8 + lax.broadcasted_iota(jnp.int32, md8.shape, 0)
        md = jnp.min(md8, axis=0, keepdims=True)
        idx_ref[pl.ds(c * CCHUNK, CCHUNK)] = jnp.min(
            jnp.where(md8 == md, code8, NUM_E), axis=0)
        psum = jnp.sum(md, axis=1, keepdims=True)
        part = psum if part is None else part + psum
    acc = jnp.broadcast_to(part, loss_ref.shape)

    @pl.when(pl.program_id(0) == 0)
    def _():
        loss_ref[...] = jnp.zeros_like(loss_ref)

    loss_ref[...] += acc

    @pl.when(pl.program_id(0) == GRID - 1)
    def _():
        loss_ref[...] = loss_ref[...] * LOSS_SCALE


_assign = pl.pallas_call(
    _assign_body,
    grid=(GRID,),
    in_specs=[
        pl.BlockSpec((ROW_BLK, DIM), lambda i: (i, 0)),
        pl.BlockSpec((NUM_E, DIM), lambda i: (0, 0)),
    ],
    out_specs=[
        pl.BlockSpec((ROW_BLK,), lambda i: (i,)),
        pl.BlockSpec((8, 128), lambda i: (0, 0)),
    ],
    out_shape=[
        jax.ShapeDtypeStruct((N_ROWS,), jnp.int32),
        jax.ShapeDtypeStruct((8, 128), jnp.float32),
    ],
    compiler_params=pltpu.CompilerParams(
        dimension_semantics=("arbitrary",),
    ),
)


@functools.cache
def _make_gather_rows():
    @functools.partial(
        pl.kernel,
        mesh=plsc.VectorSubcoreMesh(core_axis_name="c", subcore_axis_name="s"),
        out_type=jax.ShapeDtypeStruct((N_ROWS, DIM), jnp.float32),
        scratch_types=[
            pltpu.VMEM((B_PER_W,), jnp.int32),
            pltpu.VMEM((B_PER_W, DIM), jnp.float32),
            pltpu.SemaphoreType.DMA,
        ],
        compiler_params=pltpu.CompilerParams(use_tc_tiling_on_sc=False, skip_device_barrier=True),
    )
    def _gather_rows(table_hbm, idx_hbm, out_hbm, idx_v, rows_v, sem):
        wid = lax.axis_index("s") * 2 + lax.axis_index("c")
        base = wid * B_PER_W
        pltpu.sync_copy(idx_hbm.at[pl.ds(base, B_PER_W)], idx_v)
        pltpu.async_copy(table_hbm.at[idx_v], rows_v, sem).wait()
        pltpu.sync_copy(rows_v, out_hbm.at[pl.ds(base, B_PER_W)])

    return _gather_rows


def kernel(inputs, embedding):
    flat = inputs.reshape(N_ROWS, DIM)
    idx, loss_buf = _assign(flat, embedding)
    quantized = _make_gather_rows()(embedding, idx)
    return quantized.reshape(inputs.shape), loss_buf[0, 0]


# speedup vs baseline: 1.7184x; 1.0005x over previous
import functools

import jax
import jax.numpy as jnp
from jax import lax
from jax.experimental import pallas as pl
from jax.experimental.pallas import tpu as pltpu
from jax.experimental.pallas import tpu_sc as plsc

NUM_E = 1024
DIM = 64
N_ROWS = 32 * 576
ROW_BLK = 6144
CCHUNK = 512
GRID = N_ROWS // ROW_BLK
LOSS_SCALE = 1.25 / (N_ROWS * DIM)

NW = 32
B_PER_W = N_ROWS // NW
IDX_CHUNK = 72
CHUNKS_PER_W = B_PER_W // IDX_CHUNK


def _assign_body(x_ref, e_ref, idx_ref, loss_ref):
    x = x_ref[...]
    e = e_ref[...]
    xt = x.T
    mm2 = lax.dot_general(e, xt + xt, (((1,), (0,)), ((), ())),
                          preferred_element_type=jnp.float32)
    x2 = jnp.sum(xt * xt, axis=0, keepdims=True)
    e2 = jnp.sum(e * e, axis=1, keepdims=True)

    part = None
    for c in range(ROW_BLK // CCHUNK):
        cs = slice(c * CCHUNK, (c + 1) * CCHUNK)
        x2c = x2[:, cs]
        md8 = (x2c + e2[0:8]) - mm2[0:8, cs]
        iv8 = jnp.zeros(md8.shape, jnp.int32)
        for v in range(1, NUM_E // 8):
            dv = (x2c + e2[v * 8:(v + 1) * 8]) - mm2[v * 8:(v + 1) * 8, cs]
            upd = dv < md8
            md8 = jnp.minimum(md8, dv)
            iv8 = jnp.where(upd, v, iv8)

        code8 = iv8 * 8 + lax.broadcasted_iota(jnp.int32, md8.shape, 0)
        md = jnp.min(md8, axis=0, keepdims=True)
        idx_ref[pl.ds(c * CCHUNK, CCHUNK)] = jnp.min(
            jnp.where(md8 == md, code8, NUM_E), axis=0)
        psum = jnp.sum(md, axis=1, keepdims=True)
        part = psum if part is None else part + psum
    acc = jnp.broadcast_to(part, loss_ref.shape)

    @pl.when(pl.program_id(0) == 0)
    def _():
        loss_ref[...] = jnp.zeros_like(loss_ref)

    loss_ref[...] += acc

    @pl.when(pl.program_id(0) == GRID - 1)
    def _():
        loss_ref[...] = loss_ref[...] * LOSS_SCALE


_assign = pl.pallas_call(
    _assign_body,
    grid=(GRID,),
    in_specs=[
        pl.BlockSpec((ROW_BLK, DIM), lambda i: (i, 0)),
        pl.BlockSpec((NUM_E, DIM), lambda i: (0, 0)),
    ],
    out_specs=[
        pl.BlockSpec((ROW_BLK,), lambda i: (i,)),
        pl.BlockSpec((8, 128), lambda i: (0, 0)),
    ],
    out_shape=[
        jax.ShapeDtypeStruct((N_ROWS,), jnp.int32),
        jax.ShapeDtypeStruct((8, 128), jnp.float32),
    ],
    compiler_params=pltpu.CompilerParams(
        dimension_semantics=("arbitrary",),
    ),
)


@functools.cache
def _make_gather_rows():
    @functools.partial(
        pl.kernel,
        mesh=plsc.VectorSubcoreMesh(core_axis_name="c", subcore_axis_name="s"),
        out_type=jax.ShapeDtypeStruct((N_ROWS, DIM), jnp.float32),
        scratch_types=[
            pltpu.VMEM((B_PER_W,), jnp.int32),
            pltpu.VMEM((B_PER_W, DIM), jnp.float32),
            pltpu.SemaphoreType.DMA,
        ],
        compiler_params=pltpu.CompilerParams(use_tc_tiling_on_sc=False),
    )
    def _gather_rows(table_hbm, idx_hbm, out_hbm, idx_v, rows_v, sem):
        wid = lax.axis_index("s") * 2 + lax.axis_index("c")
        base = wid * B_PER_W
        pltpu.sync_copy(idx_hbm.at[pl.ds(base, B_PER_W)], idx_v)
        pltpu.async_copy(table_hbm.at[idx_v], rows_v, sem).wait()
        pltpu.sync_copy(rows_v, out_hbm.at[pl.ds(base, B_PER_W)])

    return _gather_rows


def kernel(inputs, embedding):
    flat = inputs.reshape(N_ROWS, DIM)
    idx, loss_buf = _assign(flat, embedding)
    quantized = _make_gather_rows()(embedding, idx)
    return quantized.reshape(inputs.shape), loss_buf[0, 0]
